# baseline (speedup 1.0000x reference)
import functools

import jax
import jax.numpy as jnp
from jax import lax
from jax.experimental import pallas as pl
from jax.experimental.pallas import tpu as pltpu
from jax.experimental.pallas import tpu_sc as plsc

NUM_NODES = 100000
DIM = 150
OUT_DIM = 36
HID = 144

NC = 2
NS = 16
L = 16
NW = NC * NS

B = 100000
PER_W = 3200
BP = NW * PER_W
TILES_W = PER_W // 128
CHUNK = 32
N_CHUNKS = PER_W // CHUNK
N_GROUPS = CHUNK // L

G_COL = 450
G_PAIR = {(0, 0): 0, (0, 1): 1, (0, 2): 2, (1, 1): 3, (1, 2): 4, (2, 2): 5}


GSUB = 8
GC = 250
NG = NUM_NODES // GC
GRID1 = NG // GSUB


def _ft_body(p0r, p1r, p2r, fr):
    x0, x1, x2 = p0r[...], p1r[...], p2r[...]
    for g in range(GSUB):
        a0 = jnp.transpose(x0[:, g, :], (1, 0))
        a1 = jnp.transpose(x1[:, g, :], (1, 0))
        a2 = jnp.transpose(x2[:, g, :], (1, 0))
        gs = []
        for (x, y) in [(a0, a0), (a0, a1), (a0, a2),
                       (a1, a1), (a1, a2), (a2, a2)]:
            gs.append(jnp.sum(x * y, axis=1, keepdims=True))
        wide = jnp.concatenate(
            [a0, a1, a2] + gs + [jnp.zeros((GC, 56), jnp.float32)], axis=1)
        fr[g * 4 * GC:(g + 1) * 4 * GC, :] = wide.reshape(4 * GC, 128)


def _build_ft(p0t, p1t, p2t):
    return pl.pallas_call(
        _ft_body,
        grid=(GRID1,),
        in_specs=[pl.BlockSpec((DIM, GSUB, GC), lambda i: (0, i, 0))] * 3,
        out_specs=pl.BlockSpec((4 * GSUB * GC, 128), lambda i: (i, 0)),
        out_shape=jax.ShapeDtypeStruct((4 * NUM_NODES, 128), jnp.float32),
    )(p0t, p1t, p2t)


def _sc_body(src_h, dst_h, ft, rf_h,
             idx_s, idx_d, exp_s, exp_d, sa, da, sb, db, rfb,
             sem_a, sem_b):
    cid = lax.axis_index("c")
    sid = lax.axis_index("s")
    wid = sid * NC + cid
    base = wid * PER_W

    pltpu.sync_copy(src_h.at[pl.ds(base, PER_W)], idx_s)
    pltpu.sync_copy(dst_h.at[pl.ds(base, PER_W)], idx_d)

    lanes = lax.iota(jnp.int32, L)

    @pl.loop(0, PER_W // L)
    def _expand(i):
        pos = i * L
        addr = (lanes + pos) * 4
        for (st, ex) in ((idx_s, exp_s), (idx_d, exp_d)):
            v4 = st[pl.ds(pos, L)] * 4
            for q in range(4):
                plsc.store_scatter(ex, [addr + q], v4 + q)

    set_a = (sa, da, sem_a)
    set_b = (sb, db, sem_b)

    def fire(ch, bufset):
        sbuf, dbuf, sem = bufset
        off = ch * CHUNK * 4
        pltpu.async_copy(ft.at[exp_s.at[pl.ds(off, CHUNK * 4)]], sbuf, sem)
        pltpu.async_copy(ft.at[exp_d.at[pl.ds(off, CHUNK * 4)]], dbuf, sem)

    def drain(bufset):
        sbuf, dbuf, sem = bufset
        pltpu.make_async_copy(ft.at[exp_s.at[pl.ds(0, CHUNK * 4)]],
                              sbuf, sem).wait()
        pltpu.make_async_copy(ft.at[exp_d.at[pl.ds(0, CHUNK * 4)]],
                              dbuf, sem).wait()

    zero16 = jnp.zeros((L,), jnp.int32)

    def compute(sub, bufset):
        sbuf, dbuf, _ = bufset
        for g in range(N_GROUPS):
            items = lanes + g * L
            items512 = items * 512
            accs = tuple(jnp.zeros((L,), jnp.float32) for _ in range(9))
            k0 = lanes * 9

            def kstep(_, carry):
                accs = carry[:9]
                k = carry[9]
                a0 = items512 + k
                a1 = a0 + 150
                a2 = a0 + 300
                sv = [plsc.load_gather(sbuf, [zero16, a])
                      for a in (a0, a1, a2)]
                dv = [plsc.load_gather(dbuf, [zero16, a])
                      for a in (a0, a1, a2)]
                new = tuple(accs[a * 3 + b] + sv[a] * dv[b]
                            for a in range(3) for b in range(3))
                kn = k + 1
                kn = jnp.where(kn == DIM, 0, kn)
                return new + (kn,)

            carry = lax.fori_loop(0, DIM, kstep, accs + (k0,), unroll=5)
            accs = carry[:9]

            gsv = [plsc.load_gather(sbuf, [zero16, items512 + (G_COL + j)])
                   for j in range(6)]
            gdv = [plsc.load_gather(dbuf, [zero16, items512 + (G_COL + j)])
                   for j in range(6)]
            col = sub * CHUNK + g * L
            for i in range(3):
                for j in range(3):
                    pair = G_PAIR[(min(i, j), max(i, j))]
                    rfb[i * 6 + j, pl.ds(col, L)] = gsv[pair]
                    rfb[(3 + i) * 6 + 3 + j, pl.ds(col, L)] = gdv[pair]
                    v = accs[i * 3 + j]
                    rfb[i * 6 + 3 + j, pl.ds(col, L)] = v
                    rfb[(3 + j) * 6 + i, pl.ds(col, L)] = v

    fire(0, set_a)

    @pl.loop(0, N_CHUNKS, step=4)
    def _tile(ch):
        fire(ch + 1, set_b)
        drain(set_a)
        compute(0, set_a)

        fire(ch + 2, set_a)
        drain(set_b)
        compute(1, set_b)

        fire(ch + 3, set_b)
        drain(set_a)
        compute(2, set_a)

        @pl.when(ch + 4 < N_CHUNKS)
        def _():
            fire(ch + 4, set_a)
        drain(set_b)
        compute(3, set_b)

        pltpu.sync_copy(rfb, rf_h.at[:, wid * TILES_W + ch // 4])


_sc_gram = functools.partial(
    pl.kernel,
    out_type=jax.ShapeDtypeStruct((OUT_DIM, BP // 128, 128), jnp.float32),
    mesh=plsc.VectorSubcoreMesh(core_axis_name="c", subcore_axis_name="s"),
    scratch_types=[
        pltpu.VMEM((PER_W,), jnp.int32),
        pltpu.VMEM((PER_W,), jnp.int32),
        pltpu.VMEM((PER_W * 4,), jnp.int32),
        pltpu.VMEM((PER_W * 4,), jnp.int32),
    ] + [pltpu.VMEM((CHUNK * 4, 128), jnp.float32) for _ in range(4)] + [
        pltpu.VMEM((OUT_DIM, 128), jnp.float32),
        pltpu.SemaphoreType.DMA,
        pltpu.SemaphoreType.DMA,
    ],
    compiler_params=pltpu.CompilerParams(use_tc_tiling_on_sc=False,
                                         needs_layout_passes=False,
                                         disable_bounds_checks=True),
)(_sc_body)


BT = 2048


def _mlp_body(rf_ref, w1_ref, b1_ref, w2_ref, b2_ref, out_ref):
    x = rf_ref[...].reshape(OUT_DIM, BT)
    x = jnp.log1p(jnp.maximum(x, 0.0))
    h = jnp.dot(w1_ref[...], x, preferred_element_type=jnp.float32)
    h = jnp.maximum(h + b1_ref[...], 0.0)
    o = jnp.dot(w2_ref[...], h, preferred_element_type=jnp.float32)
    out_ref[...] = o + b2_ref[...]


def _mlp(rf3, w1, b1, w2, b2):
    grid = (BP // BT,)
    return pl.pallas_call(
        _mlp_body,
        grid=grid,
        in_specs=[
            pl.BlockSpec((OUT_DIM, BT // 128, 128), lambda i: (0, i, 0)),
            pl.BlockSpec((HID, OUT_DIM), lambda i: (0, 0)),
            pl.BlockSpec((HID, 1), lambda i: (0, 0)),
            pl.BlockSpec((OUT_DIM, HID), lambda i: (0, 0)),
            pl.BlockSpec((OUT_DIM, 1), lambda i: (0, 0)),
        ],
        out_specs=pl.BlockSpec((OUT_DIM, BT), lambda i: (0, i)),
        out_shape=jax.ShapeDtypeStruct((OUT_DIM, BP), jnp.float32),
    )(rf3, w1, b1, w2, b2)


def kernel(src, dst, p0, p1, p2, w1, b1, w2, b2):
    pts = [p.T.reshape(DIM, NG, GC) for p in (p0, p1, p2)]
    ft = _build_ft(*pts)
    pad = BP - B
    src_p = jnp.concatenate([src.astype(jnp.int32),
                             jnp.zeros((pad,), jnp.int32)])
    dst_p = jnp.concatenate([dst.astype(jnp.int32),
                             jnp.zeros((pad,), jnp.int32)])
    rf3 = _sc_gram(src_p, dst_p, ft)
    out_t = _mlp(rf3, w1, b1.reshape(HID, 1), w2, b2.reshape(OUT_DIM, 1))
    return out_t[:, :B].T


# speedup vs baseline: 1.1442x; 1.1442x over previous
import functools

import jax
import jax.numpy as jnp
from jax import lax
from jax.experimental import pallas as pl
from jax.experimental.pallas import tpu as pltpu
from jax.experimental.pallas import tpu_sc as plsc

NUM_NODES = 100000
DIM = 150
OUT_DIM = 36
HID = 144

NC = 2
NS = 16
L = 16
NW = NC * NS

B = 100000
PER_W = 3200
BP = NW * PER_W
TILES_W = PER_W // 128
CHUNK = 32
N_CHUNKS = PER_W // CHUNK
N_GROUPS = CHUNK // L

G_COL = 450
G_PAIR = {(0, 0): 0, (0, 1): 1, (0, 2): 2, (1, 1): 3, (1, 2): 4, (2, 2): 5}


RN = 2000


def _ft_body(p0r, p1r, p2r, fr):
    a0, a1, a2 = p0r[...], p1r[...], p2r[...]
    gs = []
    for (x, y) in [(a0, a0), (a0, a1), (a0, a2),
                   (a1, a1), (a1, a2), (a2, a2)]:
        gs.append(jnp.sum(x * y, axis=1, keepdims=True))
    wide = jnp.concatenate(
        [a0, a1, a2] + gs + [jnp.zeros((RN, 56), jnp.float32)], axis=1)
    fr[...] = wide.reshape(4 * RN, 128)


def _build_ft(p0, p1, p2):
    return pl.pallas_call(
        _ft_body,
        grid=(NUM_NODES // RN,),
        in_specs=[pl.BlockSpec((RN, DIM), lambda i: (i, 0))] * 3,
        out_specs=pl.BlockSpec((4 * RN, 128), lambda i: (i, 0)),
        out_shape=jax.ShapeDtypeStruct((4 * NUM_NODES, 128), jnp.float32),
    )(p0, p1, p2)


def _sc_body(src_h, dst_h, ft, rf_h,
             idx_s, idx_d, exp_s, exp_d, sa, da, sb, db, rfb,
             sem_a, sem_b):
    cid = lax.axis_index("c")
    sid = lax.axis_index("s")
    wid = sid * NC + cid
    base = wid * PER_W

    pltpu.sync_copy(src_h.at[pl.ds(base, PER_W)], idx_s)
    pltpu.sync_copy(dst_h.at[pl.ds(base, PER_W)], idx_d)

    lanes = lax.iota(jnp.int32, L)

    @pl.loop(0, PER_W // L)
    def _expand(i):
        pos = i * L
        addr = (lanes + pos) * 4
        for (st, ex) in ((idx_s, exp_s), (idx_d, exp_d)):
            v4 = st[pl.ds(pos, L)] * 4
            for q in range(4):
                plsc.store_scatter(ex, [addr + q], v4 + q)

    set_a = (sa, da, sem_a)
    set_b = (sb, db, sem_b)

    NSUB = 4
    SUBR = CHUNK * 4 // NSUB

    def fire(ch, bufset):
        sbuf, dbuf, sem = bufset
        off = ch * CHUNK * 4
        for s in range(NSUB):
            pltpu.async_copy(ft.at[exp_s.at[pl.ds(off + s * SUBR, SUBR)]],
                             sbuf.at[pl.ds(s * SUBR, SUBR)], sem)
            pltpu.async_copy(ft.at[exp_d.at[pl.ds(off + s * SUBR, SUBR)]],
                             dbuf.at[pl.ds(s * SUBR, SUBR)], sem)

    def drain(bufset):
        sbuf, dbuf, sem = bufset
        for s in range(NSUB):
            pltpu.make_async_copy(ft.at[exp_s.at[pl.ds(0, SUBR)]],
                                  sbuf.at[pl.ds(s * SUBR, SUBR)], sem).wait()
            pltpu.make_async_copy(ft.at[exp_d.at[pl.ds(0, SUBR)]],
                                  dbuf.at[pl.ds(s * SUBR, SUBR)], sem).wait()

    zero16 = jnp.zeros((L,), jnp.int32)

    def compute(sub, bufset):
        sbuf, dbuf, _ = bufset
        for g in range(N_GROUPS):
            items = lanes + g * L
            items512 = items * 512
            accs = tuple(jnp.zeros((L,), jnp.float32) for _ in range(9))
            k0 = lanes * 9

            def kstep(_, carry):
                accs = carry[:9]
                k = carry[9]
                a0 = items512 + k
                a1 = a0 + 150
                a2 = a0 + 300
                sv = [plsc.load_gather(sbuf, [zero16, a])
                      for a in (a0, a1, a2)]
                dv = [plsc.load_gather(dbuf, [zero16, a])
                      for a in (a0, a1, a2)]
                new = tuple(accs[a * 3 + b] + sv[a] * dv[b]
                            for a in range(3) for b in range(3))
                kn = k + 1
                kn = jnp.where(kn == DIM, 0, kn)
                return new + (kn,)

            carry = lax.fori_loop(0, DIM, kstep, accs + (k0,), unroll=5)
            accs = carry[:9]

            gsv = [plsc.load_gather(sbuf, [zero16, items512 + (G_COL + j)])
                   for j in range(6)]
            gdv = [plsc.load_gather(dbuf, [zero16, items512 + (G_COL + j)])
                   for j in range(6)]
            col = sub * CHUNK + g * L
            for i in range(3):
                for j in range(3):
                    pair = G_PAIR[(min(i, j), max(i, j))]
                    rfb[i * 6 + j, pl.ds(col, L)] = gsv[pair]
                    rfb[(3 + i) * 6 + 3 + j, pl.ds(col, L)] = gdv[pair]
                    v = accs[i * 3 + j]
                    rfb[i * 6 + 3 + j, pl.ds(col, L)] = v
                    rfb[(3 + j) * 6 + i, pl.ds(col, L)] = v

    fire(0, set_a)

    @pl.loop(0, N_CHUNKS, step=4)
    def _tile(ch):
        fire(ch + 1, set_b)
        drain(set_a)
        compute(0, set_a)

        fire(ch + 2, set_a)
        drain(set_b)
        compute(1, set_b)

        fire(ch + 3, set_b)
        drain(set_a)
        compute(2, set_a)

        @pl.when(ch + 4 < N_CHUNKS)
        def _():
            fire(ch + 4, set_a)
        drain(set_b)
        compute(3, set_b)

        pltpu.sync_copy(rfb, rf_h.at[:, wid * TILES_W + ch // 4])


_sc_gram = functools.partial(
    pl.kernel,
    out_type=jax.ShapeDtypeStruct((OUT_DIM, BP // 128, 128), jnp.float32),
    mesh=plsc.VectorSubcoreMesh(core_axis_name="c", subcore_axis_name="s"),
    scratch_types=[
        pltpu.VMEM((PER_W,), jnp.int32),
        pltpu.VMEM((PER_W,), jnp.int32),
        pltpu.VMEM((PER_W * 4,), jnp.int32),
        pltpu.VMEM((PER_W * 4,), jnp.int32),
    ] + [pltpu.VMEM((CHUNK * 4, 128), jnp.float32) for _ in range(4)] + [
        pltpu.VMEM((OUT_DIM, 128), jnp.float32),
        pltpu.SemaphoreType.DMA,
        pltpu.SemaphoreType.DMA,
    ],
    compiler_params=pltpu.CompilerParams(use_tc_tiling_on_sc=False,
                                         needs_layout_passes=False,
                                         disable_bounds_checks=True),
)(_sc_body)


BT = 2048


def _mlp_body(rf_ref, w1_ref, b1_ref, w2_ref, b2_ref, out_ref):
    x = rf_ref[...].reshape(OUT_DIM, BT)
    x = jnp.log1p(jnp.maximum(x, 0.0))
    h = jnp.dot(w1_ref[...], x, preferred_element_type=jnp.float32)
    h = jnp.maximum(h + b1_ref[...], 0.0)
    o = jnp.dot(w2_ref[...], h, preferred_element_type=jnp.float32)
    out_ref[...] = o + b2_ref[...]


def _mlp(rf3, w1, b1, w2, b2):
    grid = (BP // BT,)
    return pl.pallas_call(
        _mlp_body,
        grid=grid,
        in_specs=[
            pl.BlockSpec((OUT_DIM, BT // 128, 128), lambda i: (0, i, 0)),
            pl.BlockSpec((HID, OUT_DIM), lambda i: (0, 0)),
            pl.BlockSpec((HID, 1), lambda i: (0, 0)),
            pl.BlockSpec((OUT_DIM, HID), lambda i: (0, 0)),
            pl.BlockSpec((OUT_DIM, 1), lambda i: (0, 0)),
        ],
        out_specs=pl.BlockSpec((OUT_DIM, BT), lambda i: (0, i)),
        out_shape=jax.ShapeDtypeStruct((OUT_DIM, BP), jnp.float32),
    )(rf3, w1, b1, w2, b2)


def kernel(src, dst, p0, p1, p2, w1, b1, w2, b2):
    ft = _build_ft(p0, p1, p2)
    pad = BP - B
    src_p = jnp.concatenate([src.astype(jnp.int32),
                             jnp.zeros((pad,), jnp.int32)])
    dst_p = jnp.concatenate([dst.astype(jnp.int32),
                             jnp.zeros((pad,), jnp.int32)])
    rf3 = _sc_gram(src_p, dst_p, ft)
    out_t = _mlp(rf3, w1, b1.reshape(HID, 1), w2, b2.reshape(OUT_DIM, 1))
    return out_t[:, :B].T
